# baseline (speedup 1.0000x reference)
import functools

import jax
import jax.numpy as jnp
from jax import lax
from jax.experimental import pallas as pl
from jax.experimental.pallas import tpu as pltpu
from jax.experimental.pallas import tpu_sc as plsc

N = 65536
M = 32768
DV = 4
DC = 8
T = 8
W = N // DC
NW = 32

NVV_SC = 0
W_TC = W - NVV_SC


TC_GRID = 4


def _make_tc_kernel(s):

    def body(betas_ref, alphas_ref, llr_ref, dec_ref, post_ref):
        llr = llr_ref[...]
        llr4 = llr[None]
        v2c = jnp.broadcast_to(llr4, (DV, DC, s, 128))
        big = jnp.float32(1e30)
        smask = jnp.int32(-2147483648)

        def signed(magnitude, sign_bits):
            i = jax.lax.bitcast_convert_type(magnitude, jnp.int32)
            return jax.lax.bitcast_convert_type(i ^ sign_bits, jnp.float32)

        for t in range(T):
            mag = jnp.abs(v2c)
            sbit = jax.lax.bitcast_convert_type(v2c, jnp.int32) & smask
            xall = sbit[:, 0]
            for k in range(1, DC):
                xall = xall ^ sbit[:, k]
            extb = xall[:, None] ^ sbit
            min1 = jnp.min(mag, axis=1, keepdims=True)
            is_min = mag <= min1
            min2 = jnp.min(jnp.where(is_min, big, mag), axis=1, keepdims=True)
            if t < T - 1:
                q = alphas_ref[t] * betas_ref[t]
            else:
                q = betas_ref[t]
            qa1 = q * min1
            qa2 = jnp.where(min1 == 0.0, 0.0, q * min2)
            ac2v = signed(jnp.where(is_min, qa2, qa1), extb)
            acc = jnp.sum(ac2v, axis=0, keepdims=True)
            if t < T - 1:
                v2c = (llr4 + acc) - ac2v
        post = llr + acc[0]
        post_ref[...] = post
        dec_ref[...] = (post < 0).astype(jnp.int32)

    return body


def _run_tc(llr2, betas, alphas, w):
    s = w // 128
    grid = TC_GRID if s % TC_GRID == 0 else 1
    bs = s // grid
    llr3 = llr2.reshape(DC, s, 128)
    dec3, post3 = pl.pallas_call(
        _make_tc_kernel(bs),
        grid=(grid,),
        out_shape=(
            jax.ShapeDtypeStruct((DC, s, 128), jnp.int32),
            jax.ShapeDtypeStruct((DC, s, 128), jnp.float32),
        ),
        in_specs=[
            pl.BlockSpec(memory_space=pltpu.SMEM),
            pl.BlockSpec(memory_space=pltpu.SMEM),
            pl.BlockSpec((DC, bs, 128), lambda i: (0, i, 0)),
        ],
        out_specs=(
            pl.BlockSpec((DC, bs, 128), lambda i: (0, i, 0)),
            pl.BlockSpec((DC, bs, 128), lambda i: (0, i, 0)),
        ),
    )(betas, alphas, llr3)
    return dec3.reshape(DC, w), post3.reshape(DC, w)


def _make_sc_body(chunk):
    lg = chunk // 16

    def body(llr_hbm, betas_hbm, alphas_hbm, dec_hbm, post_hbm,
             llr_v, bet_v, alp_v, post_v, dec_v):
        wid = lax.axis_index("s") * 2 + lax.axis_index("c")
        pltpu.sync_copy(llr_hbm.at[wid], llr_v)
        pltpu.sync_copy(betas_hbm, bet_v)
        pltpu.sync_copy(alphas_hbm, alp_v)

        def group(g, carry):
            ds = pl.ds(g * 16, 16)
            llr_g = [llr_v[k, ds] for k in range(DC)]
            msg = [[llr_g[k] for k in range(DC)] for _ in range(DV)]
            for t in range(T):
                beta = bet_v[t, :]
                alpha = alp_v[t, :]
                for j in range(DV):
                    row = msg[j]
                    sgn = [jnp.sign(x) for x in row]
                    mag = [jnp.abs(x) for x in row]
                    ts = sgn[0]
                    for k in range(1, DC):
                        ts = ts * sgn[k]
                    m1 = mag[0]
                    for k in range(1, DC):
                        m1 = jnp.minimum(m1, mag[k])
                    big = jnp.full((16,), 1e30, dtype=jnp.float32)
                    m2 = jnp.minimum(jnp.where(mag[0] <= m1, big, mag[0]),
                                     jnp.where(mag[1] <= m1, big, mag[1]))
                    for k in range(2, DC):
                        m2 = jnp.minimum(m2,
                                         jnp.where(mag[k] <= m1, big, mag[k]))
                    for k in range(DC):
                        ext = jnp.where(mag[k] <= m1, m2, m1)
                        row[k] = beta * ext * (ts * sgn[k])
                if t < T - 1:
                    for k in range(DC):
                        s = msg[0][k] + msg[1][k] + msg[2][k] + msg[3][k]
                        lk = llr_g[k]
                        for j in range(DV):
                            msg[j][k] = lk + alpha * (s - msg[j][k])
            for k in range(DC):
                p = llr_g[k] + msg[0][k] + msg[1][k] + msg[2][k] + msg[3][k]
                post_v[k, ds] = p
                dec_v[k, ds] = jnp.where(p < 0.0, 1, 0).astype(jnp.int32)
            return carry

        lax.fori_loop(0, lg, group, None)
        pltpu.sync_copy(post_v, post_hbm.at[wid])
        pltpu.sync_copy(dec_v, dec_hbm.at[wid])

    return body


def _run_sc(llr2_sc, betas, alphas, nvv):
    chunk = nvv // NW
    llr3 = llr2_sc.reshape(DC, NW, chunk).transpose(1, 0, 2)
    bet = jnp.broadcast_to(betas[:, None], (T, 16))
    alp = jnp.broadcast_to(alphas[:, None], (T, 16))
    run = functools.partial(
        pl.kernel,
        out_type=(
            jax.ShapeDtypeStruct((NW, DC, chunk), jnp.int32),
            jax.ShapeDtypeStruct((NW, DC, chunk), jnp.float32),
        ),
        mesh=plsc.VectorSubcoreMesh(core_axis_name="c", subcore_axis_name="s"),
        scratch_types=[
            pltpu.VMEM((DC, chunk), jnp.float32),
            pltpu.VMEM((T, 16), jnp.float32),
            pltpu.VMEM((T, 16), jnp.float32),
            pltpu.VMEM((DC, chunk), jnp.float32),
            pltpu.VMEM((DC, chunk), jnp.int32),
        ],
    )(_make_sc_body(chunk))
    dec3, post3 = run(llr3, bet, alp)
    return (dec3.transpose(1, 0, 2).reshape(DC, nvv),
            post3.transpose(1, 0, 2).reshape(DC, nvv))


def kernel(llr, betas, alphas):
    llr2 = llr.reshape(DC, W)
    if NVV_SC == 0:
        dec2, post2 = _run_tc(llr2, betas, alphas, W)
    elif W_TC == 0:
        dec2, post2 = _run_sc(llr2, betas, alphas, W)
    else:
        dec_tc, post_tc = _run_tc(llr2[:, :W_TC], betas, alphas, W_TC)
        dec_sc, post_sc = _run_sc(llr2[:, W_TC:], betas, alphas, NVV_SC)
        dec2 = jnp.concatenate([dec_tc, dec_sc], axis=1)
        post2 = jnp.concatenate([post_tc, post_sc], axis=1)
    return dec2.reshape(N), post2.reshape(N)


# speedup vs baseline: 1.2469x; 1.2469x over previous
import functools

import jax
import jax.numpy as jnp
from jax import lax
from jax.experimental import pallas as pl
from jax.experimental.pallas import tpu as pltpu
from jax.experimental.pallas import tpu_sc as plsc

N = 65536
M = 32768
DV = 4
DC = 8
T = 8
W = N // DC
NW = 32

NVV_SC = 0
W_TC = W - NVV_SC


TC_GRID = 1


def _make_tc_kernel(s):

    def body(betas_ref, alphas_ref, llr_ref, dec_ref, post_ref):
        llr = llr_ref[...]
        llr4 = llr[None]
        v2c = jnp.broadcast_to(llr4, (DV, DC, s, 128))
        big = jnp.float32(1e30)
        smask = jnp.int32(-2147483648)

        def signed(magnitude, sign_bits):
            i = jax.lax.bitcast_convert_type(magnitude, jnp.int32)
            return jax.lax.bitcast_convert_type(i ^ sign_bits, jnp.float32)

        for t in range(T):
            mag = jnp.abs(v2c)
            sbit = jax.lax.bitcast_convert_type(v2c, jnp.int32) & smask
            xall = sbit[:, 0]
            for k in range(1, DC):
                xall = xall ^ sbit[:, k]
            extb = xall[:, None] ^ sbit
            min1 = jnp.min(mag, axis=1, keepdims=True)
            is_min = mag <= min1
            min2 = jnp.min(jnp.where(is_min, big, mag), axis=1, keepdims=True)
            if t < T - 1:
                q = alphas_ref[t] * betas_ref[t]
            else:
                q = betas_ref[t]
            qa1 = q * min1
            qa2 = jnp.where(min1 == 0.0, 0.0, q * min2)
            ac2v = signed(jnp.where(is_min, qa2, qa1), extb)
            acc = jnp.sum(ac2v, axis=0, keepdims=True)
            if t < T - 1:
                v2c = (llr4 + acc) - ac2v
        post = llr + acc[0]
        post_ref[...] = post
        dec_ref[...] = (post < 0).astype(jnp.int32)

    return body


def _run_tc(llr2, betas, alphas, w):
    s = w // 128
    grid = TC_GRID if s % TC_GRID == 0 else 1
    bs = s // grid
    llr3 = llr2.reshape(DC, s, 128)
    dec3, post3 = pl.pallas_call(
        _make_tc_kernel(bs),
        grid=(grid,),
        out_shape=(
            jax.ShapeDtypeStruct((DC, s, 128), jnp.int32),
            jax.ShapeDtypeStruct((DC, s, 128), jnp.float32),
        ),
        in_specs=[
            pl.BlockSpec(memory_space=pltpu.SMEM),
            pl.BlockSpec(memory_space=pltpu.SMEM),
            pl.BlockSpec((DC, bs, 128), lambda i: (0, i, 0)),
        ],
        out_specs=(
            pl.BlockSpec((DC, bs, 128), lambda i: (0, i, 0)),
            pl.BlockSpec((DC, bs, 128), lambda i: (0, i, 0)),
        ),
    )(betas, alphas, llr3)
    return dec3.reshape(DC, w), post3.reshape(DC, w)


def _make_sc_body(chunk):
    lg = chunk // 16

    def body(llr_hbm, betas_hbm, alphas_hbm, dec_hbm, post_hbm,
             llr_v, bet_v, alp_v, post_v, dec_v):
        wid = lax.axis_index("s") * 2 + lax.axis_index("c")
        pltpu.sync_copy(llr_hbm.at[wid], llr_v)
        pltpu.sync_copy(betas_hbm, bet_v)
        pltpu.sync_copy(alphas_hbm, alp_v)

        def group(g, carry):
            ds = pl.ds(g * 16, 16)
            llr_g = [llr_v[k, ds] for k in range(DC)]
            msg = [[llr_g[k] for k in range(DC)] for _ in range(DV)]
            for t in range(T):
                beta = bet_v[t, :]
                alpha = alp_v[t, :]
                for j in range(DV):
                    row = msg[j]
                    sgn = [jnp.sign(x) for x in row]
                    mag = [jnp.abs(x) for x in row]
                    ts = sgn[0]
                    for k in range(1, DC):
                        ts = ts * sgn[k]
                    m1 = mag[0]
                    for k in range(1, DC):
                        m1 = jnp.minimum(m1, mag[k])
                    big = jnp.full((16,), 1e30, dtype=jnp.float32)
                    m2 = jnp.minimum(jnp.where(mag[0] <= m1, big, mag[0]),
                                     jnp.where(mag[1] <= m1, big, mag[1]))
                    for k in range(2, DC):
                        m2 = jnp.minimum(m2,
                                         jnp.where(mag[k] <= m1, big, mag[k]))
                    for k in range(DC):
                        ext = jnp.where(mag[k] <= m1, m2, m1)
                        row[k] = beta * ext * (ts * sgn[k])
                if t < T - 1:
                    for k in range(DC):
                        s = msg[0][k] + msg[1][k] + msg[2][k] + msg[3][k]
                        lk = llr_g[k]
                        for j in range(DV):
                            msg[j][k] = lk + alpha * (s - msg[j][k])
            for k in range(DC):
                p = llr_g[k] + msg[0][k] + msg[1][k] + msg[2][k] + msg[3][k]
                post_v[k, ds] = p
                dec_v[k, ds] = jnp.where(p < 0.0, 1, 0).astype(jnp.int32)
            return carry

        lax.fori_loop(0, lg, group, None)
        pltpu.sync_copy(post_v, post_hbm.at[wid])
        pltpu.sync_copy(dec_v, dec_hbm.at[wid])

    return body


def _run_sc(llr2_sc, betas, alphas, nvv):
    chunk = nvv // NW
    llr3 = llr2_sc.reshape(DC, NW, chunk).transpose(1, 0, 2)
    bet = jnp.broadcast_to(betas[:, None], (T, 16))
    alp = jnp.broadcast_to(alphas[:, None], (T, 16))
    run = functools.partial(
        pl.kernel,
        out_type=(
            jax.ShapeDtypeStruct((NW, DC, chunk), jnp.int32),
            jax.ShapeDtypeStruct((NW, DC, chunk), jnp.float32),
        ),
        mesh=plsc.VectorSubcoreMesh(core_axis_name="c", subcore_axis_name="s"),
        scratch_types=[
            pltpu.VMEM((DC, chunk), jnp.float32),
            pltpu.VMEM((T, 16), jnp.float32),
            pltpu.VMEM((T, 16), jnp.float32),
            pltpu.VMEM((DC, chunk), jnp.float32),
            pltpu.VMEM((DC, chunk), jnp.int32),
        ],
    )(_make_sc_body(chunk))
    dec3, post3 = run(llr3, bet, alp)
    return (dec3.transpose(1, 0, 2).reshape(DC, nvv),
            post3.transpose(1, 0, 2).reshape(DC, nvv))


def kernel(llr, betas, alphas):
    llr2 = llr.reshape(DC, W)
    if NVV_SC == 0:
        dec2, post2 = _run_tc(llr2, betas, alphas, W)
    elif W_TC == 0:
        dec2, post2 = _run_sc(llr2, betas, alphas, W)
    else:
        dec_tc, post_tc = _run_tc(llr2[:, :W_TC], betas, alphas, W_TC)
        dec_sc, post_sc = _run_sc(llr2[:, W_TC:], betas, alphas, NVV_SC)
        dec2 = jnp.concatenate([dec_tc, dec_sc], axis=1)
        post2 = jnp.concatenate([post_tc, post_sc], axis=1)
    return dec2.reshape(N), post2.reshape(N)


# speedup vs baseline: 1.3494x; 1.0822x over previous
import functools

import jax
import jax.numpy as jnp
from jax import lax
from jax.experimental import pallas as pl
from jax.experimental.pallas import tpu as pltpu
from jax.experimental.pallas import tpu_sc as plsc

N = 65536
M = 32768
DV = 4
DC = 8
T = 8
W = N // DC
NW = 32

NVV_SC = 0
W_TC = W - NVV_SC


TC_GRID = 1


def _make_tc_kernel(s):

    def body(betas_ref, alphas_ref, llr_ref, dec_ref, post_ref):
        llr = llr_ref[...]
        v2c = llr
        big = jnp.float32(1e30)
        smask = jnp.int32(-2147483648)

        def signed(magnitude, sign_bits):
            i = jax.lax.bitcast_convert_type(magnitude, jnp.int32)
            return jax.lax.bitcast_convert_type(i ^ sign_bits, jnp.float32)

        for t in range(T):
            mag = jnp.abs(v2c)
            sbit = jax.lax.bitcast_convert_type(v2c, jnp.int32) & smask
            xall = sbit[0]
            for k in range(1, DC):
                xall = xall ^ sbit[k]
            extb = xall[None] ^ sbit
            min1 = jnp.min(mag, axis=0, keepdims=True)
            is_min = mag <= min1
            min2 = jnp.min(jnp.where(is_min, big, mag), axis=0, keepdims=True)
            if t < T - 1:
                q = jnp.float32(DV - 1) * alphas_ref[t] * betas_ref[t]
            else:
                q = jnp.float32(DV) * betas_ref[t]
            qa1 = q * min1
            qa2 = jnp.where(min1 == 0.0, 0.0, q * min2)
            upd = signed(jnp.where(is_min, qa2, qa1), extb)
            v2c = llr + upd
        post = v2c
        post_ref[...] = post
        dec_ref[...] = (post < 0).astype(jnp.int32)

    return body


def _run_tc(llr2, betas, alphas, w):
    s = w // 128
    grid = TC_GRID if s % TC_GRID == 0 else 1
    bs = s // grid
    llr3 = llr2.reshape(DC, s, 128)
    dec3, post3 = pl.pallas_call(
        _make_tc_kernel(bs),
        grid=(grid,),
        out_shape=(
            jax.ShapeDtypeStruct((DC, s, 128), jnp.int32),
            jax.ShapeDtypeStruct((DC, s, 128), jnp.float32),
        ),
        in_specs=[
            pl.BlockSpec(memory_space=pltpu.SMEM),
            pl.BlockSpec(memory_space=pltpu.SMEM),
            pl.BlockSpec((DC, bs, 128), lambda i: (0, i, 0)),
        ],
        out_specs=(
            pl.BlockSpec((DC, bs, 128), lambda i: (0, i, 0)),
            pl.BlockSpec((DC, bs, 128), lambda i: (0, i, 0)),
        ),
    )(betas, alphas, llr3)
    return dec3.reshape(DC, w), post3.reshape(DC, w)


def _make_sc_body(chunk):
    lg = chunk // 16

    def body(llr_hbm, betas_hbm, alphas_hbm, dec_hbm, post_hbm,
             llr_v, bet_v, alp_v, post_v, dec_v):
        wid = lax.axis_index("s") * 2 + lax.axis_index("c")
        pltpu.sync_copy(llr_hbm.at[wid], llr_v)
        pltpu.sync_copy(betas_hbm, bet_v)
        pltpu.sync_copy(alphas_hbm, alp_v)

        def group(g, carry):
            ds = pl.ds(g * 16, 16)
            llr_g = [llr_v[k, ds] for k in range(DC)]
            msg = [[llr_g[k] for k in range(DC)] for _ in range(DV)]
            for t in range(T):
                beta = bet_v[t, :]
                alpha = alp_v[t, :]
                for j in range(DV):
                    row = msg[j]
                    sgn = [jnp.sign(x) for x in row]
                    mag = [jnp.abs(x) for x in row]
                    ts = sgn[0]
                    for k in range(1, DC):
                        ts = ts * sgn[k]
                    m1 = mag[0]
                    for k in range(1, DC):
                        m1 = jnp.minimum(m1, mag[k])
                    big = jnp.full((16,), 1e30, dtype=jnp.float32)
                    m2 = jnp.minimum(jnp.where(mag[0] <= m1, big, mag[0]),
                                     jnp.where(mag[1] <= m1, big, mag[1]))
                    for k in range(2, DC):
                        m2 = jnp.minimum(m2,
                                         jnp.where(mag[k] <= m1, big, mag[k]))
                    for k in range(DC):
                        ext = jnp.where(mag[k] <= m1, m2, m1)
                        row[k] = beta * ext * (ts * sgn[k])
                if t < T - 1:
                    for k in range(DC):
                        s = msg[0][k] + msg[1][k] + msg[2][k] + msg[3][k]
                        lk = llr_g[k]
                        for j in range(DV):
                            msg[j][k] = lk + alpha * (s - msg[j][k])
            for k in range(DC):
                p = llr_g[k] + msg[0][k] + msg[1][k] + msg[2][k] + msg[3][k]
                post_v[k, ds] = p
                dec_v[k, ds] = jnp.where(p < 0.0, 1, 0).astype(jnp.int32)
            return carry

        lax.fori_loop(0, lg, group, None)
        pltpu.sync_copy(post_v, post_hbm.at[wid])
        pltpu.sync_copy(dec_v, dec_hbm.at[wid])

    return body


def _run_sc(llr2_sc, betas, alphas, nvv):
    chunk = nvv // NW
    llr3 = llr2_sc.reshape(DC, NW, chunk).transpose(1, 0, 2)
    bet = jnp.broadcast_to(betas[:, None], (T, 16))
    alp = jnp.broadcast_to(alphas[:, None], (T, 16))
    run = functools.partial(
        pl.kernel,
        out_type=(
            jax.ShapeDtypeStruct((NW, DC, chunk), jnp.int32),
            jax.ShapeDtypeStruct((NW, DC, chunk), jnp.float32),
        ),
        mesh=plsc.VectorSubcoreMesh(core_axis_name="c", subcore_axis_name="s"),
        scratch_types=[
            pltpu.VMEM((DC, chunk), jnp.float32),
            pltpu.VMEM((T, 16), jnp.float32),
            pltpu.VMEM((T, 16), jnp.float32),
            pltpu.VMEM((DC, chunk), jnp.float32),
            pltpu.VMEM((DC, chunk), jnp.int32),
        ],
    )(_make_sc_body(chunk))
    dec3, post3 = run(llr3, bet, alp)
    return (dec3.transpose(1, 0, 2).reshape(DC, nvv),
            post3.transpose(1, 0, 2).reshape(DC, nvv))


def kernel(llr, betas, alphas):
    llr2 = llr.reshape(DC, W)
    if NVV_SC == 0:
        dec2, post2 = _run_tc(llr2, betas, alphas, W)
    elif W_TC == 0:
        dec2, post2 = _run_sc(llr2, betas, alphas, W)
    else:
        dec_tc, post_tc = _run_tc(llr2[:, :W_TC], betas, alphas, W_TC)
        dec_sc, post_sc = _run_sc(llr2[:, W_TC:], betas, alphas, NVV_SC)
        dec2 = jnp.concatenate([dec_tc, dec_sc], axis=1)
        post2 = jnp.concatenate([post_tc, post_sc], axis=1)
    return dec2.reshape(N), post2.reshape(N)
